# baseline (speedup 1.0000x reference)
import functools
import numpy as np
import jax
import jax.numpy as jnp
from jax import lax
from jax.experimental import pallas as pl
from jax.experimental.pallas import tpu as pltpu
from jax.experimental.pallas import tpu_sc as plsc

R, C = 32, 1000000
RG = 8
RGN = R // RG
BLK = 50176
NBLK = (C + BLK - 1) // BLK
TW = 512
NT = BLK // TW

_K0 = np.uint32(928981903)
_K1 = np.uint32(3453687069)
_K2 = np.uint32(_K0 ^ _K1 ^ np.uint32(0x1BD11BDA))
_TINY = np.float32(np.finfo(np.float32).tiny)


def _threefry_bits(x1):
    ks = (jnp.uint32(_K0), jnp.uint32(_K1), jnp.uint32(_K2))
    x0 = jnp.full_like(x1, ks[0])
    rots = ((13, 15, 26, 6), (17, 29, 16, 24))
    for i in range(5):
        for d in rots[i % 2]:
            x0 = x0 + x1
            x1 = ((x1 << jnp.uint32(d)) | (x1 >> jnp.uint32(32 - d))) ^ x0
        x0 = x0 + ks[(i + 1) % 3]
        x1 = x1 + ks[(i + 2) % 3] + jnp.uint32(i + 1)
    return x0 ^ x1


def _tc_body(xc_ref, oh_ref, m_ref, idx_ref, idxp_ref):
    rg = pl.program_id(0)
    j = pl.program_id(1)

    @pl.when(j == 0)
    def _roll():
        idxp_ref[...] = idx_ref[...]
        m_ref[...] = jnp.full((RG, 128), -jnp.inf, jnp.float32)
        idx_ref[...] = jnp.zeros((RG, 128), jnp.int32)

    @pl.when(rg < RGN)
    def _reduce():
        lane_t = jax.lax.broadcasted_iota(jnp.int32, (RG, TW), 1)
        srow = jax.lax.broadcasted_iota(jnp.int32, (RG, TW), 0)
        row_ck = ((rg * RG + srow) * C).astype(jnp.uint32) + jnp.uint32(_K1)
        g_acc = jnp.full((RG, TW), -jnp.inf, jnp.float32)
        i_acc = jnp.zeros((RG, TW), jnp.int32)
        for t in range(NT):
            colt = j * BLK + t * TW + lane_t
            x = xc_ref[:, t * TW:(t + 1) * TW]
            if (NBLK - 1) * BLK + (t + 1) * TW > C:
                x = jnp.where(colt < C, x, -jnp.inf)
            bits = _threefry_bits(row_ck + colt.astype(jnp.uint32))
            fl = jax.lax.bitcast_convert_type(
                (bits >> jnp.uint32(9)) | jnp.uint32(0x3F800000), jnp.float32)
            u = jnp.maximum(fl - jnp.float32(1.0), _TINY)
            gum = -jnp.log(-jnp.log(u))
            g = x + gum
            upd = g > g_acc
            g_acc = jnp.maximum(g_acc, g)
            i_acc = jnp.where(upd, colt, i_acc)

        bm = jnp.max(g_acc, axis=1, keepdims=True)
        cand = jnp.where(g_acc == bm, i_acc, jnp.int32(2**30))
        bidx = jnp.min(cand, axis=1, keepdims=True)

        bm_f = jnp.broadcast_to(bm, (RG, 128))
        bidx_f = jnp.broadcast_to(bidx, (RG, 128))
        upd = bm_f > m_ref[...]
        m_ref[...] = jnp.where(upd, bm_f, m_ref[...])
        idx_ref[...] = jnp.where(upd, bidx_f, idx_ref[...])

    @pl.when(rg > 0)
    def _write():
        lane = jax.lax.broadcasted_iota(jnp.int32, (RG, BLK), 1)
        oh_ref[...] = jnp.where(lane == idxp_ref[:, 0:1] - j * BLK,
                                jnp.float32(1.0), jnp.float32(0.0))


def _tc_one_hot(logits):
    grid = (RGN + 1, NBLK)
    return pl.pallas_call(
        _tc_body,
        grid=grid,
        in_specs=[
            pl.BlockSpec((RG, BLK),
                         lambda rg, j: (jnp.minimum(rg, RGN - 1),
                                        jnp.where(rg < RGN, j, NBLK - 1))),
        ],
        out_specs=[
            pl.BlockSpec((RG, BLK),
                         lambda rg, j: (jnp.maximum(rg - 1, 0),
                                        jnp.where(rg > 0, j, 0))),
        ],
        out_shape=[jax.ShapeDtypeStruct((R, C), jnp.float32)],
        scratch_shapes=[
            pltpu.VMEM((RG, 128), jnp.float32),
            pltpu.VMEM((RG, 128), jnp.int32),
            pltpu.VMEM((RG, 128), jnp.int32),
        ],
        compiler_params=pltpu.CompilerParams(
            dimension_semantics=("arbitrary", "arbitrary"),
        ),
    )(logits)[0]


_CHB = 6400
_NFULL = C // _CHB
_SCEND = (C // 128) * 128
_TAILO = _NFULL * _CHB
_TAILW = _SCEND - _TAILO
_TAILV = _TAILW // 16
_UNR = 8

_SC_MESH = dict(core_axis_name="c", subcore_axis_name="s")


def _stripe_bounds(s):
    lo = jnp.where(s < 4, 20 * s, 19 * s + 4)
    hi = jnp.where(s < 3, 20 * (s + 1), 19 * (s + 1) + 4)
    return lo, hi


def _row_exp_sum(buf, r, nvec, acc, unr=_UNR):
    assert nvec % unr == 0
    def step(i, a):
        for u in range(unr):
            a = a + jnp.exp(buf[r, pl.ds(i * (16 * unr) + u * 16, 16)])
        return a
    return lax.fori_loop(0, nvec // unr, step, acc)


def _sc_partial_sums(logits):
    @functools.partial(
        pl.kernel, mesh=plsc.VectorSubcoreMesh(**_SC_MESH),
        out_type=jax.ShapeDtypeStruct((RGN, 8, 8, 16), jnp.float32),
        scratch_types=[
            pltpu.VMEM((RG, _CHB), jnp.float32),
            pltpu.VMEM((RG, 16), jnp.float32),
        ],
    )
    def ka(x_hbm, sums_hbm, buf, accv):
        w = lax.axis_index("s") * 2 + lax.axis_index("c")
        g = w // 8
        s = w % 8
        lo, hi = _stripe_bounds(s)

        def chunk(c, accs):
            pltpu.sync_copy(
                x_hbm.at[pl.ds(g * RG, RG), pl.ds(c * _CHB, _CHB)], buf)
            return tuple(_row_exp_sum(buf, r, _CHB // 16, accs[r])
                         for r in range(RG))

        accs = lax.fori_loop(
            lo, hi, chunk,
            tuple(jnp.zeros((16,), jnp.float32) for _ in range(RG)))

        @pl.when(s == 7)
        def _tail():
            pltpu.sync_copy(
                x_hbm.at[pl.ds(g * RG, RG), pl.ds(_TAILO, _TAILW)],
                buf.at[:, pl.ds(0, _TAILW)])
            for r in range(RG):
                accv[r] = _row_exp_sum(buf, r, _TAILV, accs[r])

        @pl.when(s != 7)
        def _notail():
            for r in range(RG):
                accv[r] = accs[r]

        pltpu.sync_copy(accv, sums_hbm.at[g, s])

    return ka(logits)


def _sc_pi_write(logits, recs_arr):
    @functools.partial(
        pl.kernel, mesh=plsc.VectorSubcoreMesh(**_SC_MESH),
        out_type=jax.ShapeDtypeStruct((R, C), jnp.float32),
        scratch_types=[
            pltpu.VMEM((RG, _CHB), jnp.float32),
            pltpu.VMEM((RG, _CHB), jnp.float32),
            pltpu.VMEM((RG, 128), jnp.float32),
        ],
    )
    def kb(x_hbm, recs_hbm, pi_hbm, buf, obuf, recv):
        w = lax.axis_index("s") * 2 + lax.axis_index("c")
        g = w // 8
        s = w % 8
        pltpu.sync_copy(recs_hbm.at[pl.ds(g * RG, RG)], recv)
        recs = [recv[r, pl.ds(0, 16)] for r in range(RG)]

        def norm_rows(nvec, unr=_UNR):
            assert nvec % unr == 0
            def step(i, carry):
                for u in range(unr):
                    off = i * (16 * unr) + u * 16
                    for r in range(RG):
                        obuf[r, pl.ds(off, 16)] = (
                            jnp.exp(buf[r, pl.ds(off, 16)]) * recs[r])
                return carry
            lax.fori_loop(0, nvec // unr, step, 0)

        lo, hi = _stripe_bounds(s)

        def chunk(c, carry):
            src = x_hbm.at[pl.ds(g * RG, RG), pl.ds(c * _CHB, _CHB)]
            pltpu.sync_copy(src, buf)
            norm_rows(_CHB // 16)
            pltpu.sync_copy(
                obuf, pi_hbm.at[pl.ds(g * RG, RG), pl.ds(c * _CHB, _CHB)])
            return carry

        lax.fori_loop(lo, hi, chunk, 0)

        @pl.when(s == 7)
        def _tail():
            pltpu.sync_copy(
                x_hbm.at[pl.ds(g * RG, RG), pl.ds(_TAILO, _TAILW)],
                buf.at[:, pl.ds(0, _TAILW)])
            norm_rows(_TAILV)
            pltpu.sync_copy(
                obuf.at[:, pl.ds(0, _TAILW)],
                pi_hbm.at[pl.ds(g * RG, RG), pl.ds(_TAILO, _TAILW)])

    return kb(logits, recs_arr)


def _tc_recs_body(x_ref, sums_ref, rec_ref):
    lane = jax.lax.broadcasted_iota(jnp.int32, (R, 128), 1)
    xm = jnp.where(lane < C - _SCEND, x_ref[...], -jnp.inf)
    t64 = jnp.sum(jnp.exp(xm), axis=1, keepdims=True)
    parts = []
    for g in range(RGN):
        tg = sums_ref[g, 0]
        for s in range(1, 8):
            tg = tg + sums_ref[g, s]
        parts.append(jnp.sum(tg, axis=1, keepdims=True))
    tot = jnp.concatenate(parts, axis=0) + t64
    rec_ref[...] = jnp.broadcast_to(jnp.float32(1.0) / tot, (R, 128))


def _tc_recs(logits, sums):
    blk = _SCEND // 128
    return pl.pallas_call(
        _tc_recs_body,
        grid=(1,),
        in_specs=[
            pl.BlockSpec((R, 128), lambda i: (0, blk)),
            pl.BlockSpec((RGN, 8, 8, 16), lambda i: (0, 0, 0, 0)),
        ],
        out_specs=[pl.BlockSpec((R, 128), lambda i: (0, 0))],
        out_shape=[jax.ShapeDtypeStruct((R, 128), jnp.float32)],
    )(logits, sums)[0]


def _tc_tail_body(x_ref, rec_ref, pia_ref, pi_ref):
    del pia_ref
    lane = jax.lax.broadcasted_iota(jnp.int32, (R, 128), 1)
    xm = jnp.where(lane < C - _SCEND, x_ref[...], -jnp.inf)
    pi_ref[...] = jnp.exp(xm) * rec_ref[:, 0:1]


def _tc_tail_fix(logits, recs, pi_bulk):
    blk = _SCEND // 128
    return pl.pallas_call(
        _tc_tail_body,
        grid=(1,),
        in_specs=[
            pl.BlockSpec((R, 128), lambda i: (0, blk)),
            pl.BlockSpec((R, 128), lambda i: (0, 0)),
            pl.BlockSpec((R, 128), lambda i: (0, blk)),
        ],
        out_specs=[pl.BlockSpec((R, 128), lambda i: (0, blk))],
        out_shape=[jax.ShapeDtypeStruct((R, C), jnp.float32)],
        input_output_aliases={2: 0},
    )(logits, recs, pi_bulk)[0]


def _sc_pi_kernel(logits):
    sums = _sc_partial_sums(logits)
    recs = _tc_recs(logits, sums)
    pi_bulk = _sc_pi_write(logits, recs)
    return _tc_tail_fix(logits, recs, pi_bulk)


def kernel(logits):
    one_hot = _tc_one_hot(logits)
    pi = _sc_pi_kernel(logits)
    return (one_hot, pi)


# speedup vs baseline: 1.0027x; 1.0027x over previous
import functools
import numpy as np
import jax
import jax.numpy as jnp
from jax import lax
from jax.experimental import pallas as pl
from jax.experimental.pallas import tpu as pltpu
from jax.experimental.pallas import tpu_sc as plsc

R, C = 32, 1000000
RG = 8
RGN = R // RG
BLK = 50176
NBLK = (C + BLK - 1) // BLK
TW = 512
NT = BLK // TW

_K0 = np.uint32(928981903)
_K1 = np.uint32(3453687069)
_K2 = np.uint32(_K0 ^ _K1 ^ np.uint32(0x1BD11BDA))
_TINY = np.float32(np.finfo(np.float32).tiny)


def _threefry_bits(x1):
    ks = (jnp.uint32(_K0), jnp.uint32(_K1), jnp.uint32(_K2))
    x0 = jnp.full_like(x1, ks[0])
    rots = ((13, 15, 26, 6), (17, 29, 16, 24))
    for i in range(5):
        for d in rots[i % 2]:
            x0 = x0 + x1
            x1 = ((x1 << jnp.uint32(d)) | (x1 >> jnp.uint32(32 - d))) ^ x0
        x0 = x0 + ks[(i + 1) % 3]
        x1 = x1 + ks[(i + 2) % 3] + jnp.uint32(i + 1)
    return x0 ^ x1


def _tc_body(xc_ref, oh_ref, m_ref, idx_ref, idxp_ref):
    rg = pl.program_id(0)
    j = pl.program_id(1)

    @pl.when(j == 0)
    def _roll():
        idxp_ref[...] = idx_ref[...]
        m_ref[...] = jnp.full((RG, 128), -jnp.inf, jnp.float32)
        idx_ref[...] = jnp.zeros((RG, 128), jnp.int32)

    @pl.when(rg < RGN)
    def _reduce():
        lane_t = jax.lax.broadcasted_iota(jnp.int32, (RG, TW), 1)
        srow = jax.lax.broadcasted_iota(jnp.int32, (RG, TW), 0)
        row_ck = ((rg * RG + srow) * C).astype(jnp.uint32) + jnp.uint32(_K1)
        g_acc = jnp.full((RG, TW), -jnp.inf, jnp.float32)
        i_acc = jnp.zeros((RG, TW), jnp.int32)
        for t in range(NT):
            colt = j * BLK + t * TW + lane_t
            x = xc_ref[:, t * TW:(t + 1) * TW]
            if (NBLK - 1) * BLK + (t + 1) * TW > C:
                x = jnp.where(colt < C, x, -jnp.inf)
            bits = _threefry_bits(row_ck + colt.astype(jnp.uint32))
            fl = jax.lax.bitcast_convert_type(
                (bits >> jnp.uint32(9)) | jnp.uint32(0x3F800000), jnp.float32)
            u = jnp.maximum(fl - jnp.float32(1.0), _TINY)
            gum = -jnp.log(-jnp.log(u))
            g = x + gum
            upd = g > g_acc
            g_acc = jnp.maximum(g_acc, g)
            i_acc = jnp.where(upd, colt, i_acc)

        bm = jnp.max(g_acc, axis=1, keepdims=True)
        cand = jnp.where(g_acc == bm, i_acc, jnp.int32(2**30))
        bidx = jnp.min(cand, axis=1, keepdims=True)

        bm_f = jnp.broadcast_to(bm, (RG, 128))
        bidx_f = jnp.broadcast_to(bidx, (RG, 128))
        upd = bm_f > m_ref[...]
        m_ref[...] = jnp.where(upd, bm_f, m_ref[...])
        idx_ref[...] = jnp.where(upd, bidx_f, idx_ref[...])

    @pl.when(rg > 0)
    def _write():
        lane = jax.lax.broadcasted_iota(jnp.int32, (RG, BLK), 1)
        oh_ref[...] = jnp.where(lane == idxp_ref[:, 0:1] - j * BLK,
                                jnp.float32(1.0), jnp.float32(0.0))


def _tc_one_hot(logits):
    grid = (RGN + 1, NBLK)
    return pl.pallas_call(
        _tc_body,
        grid=grid,
        in_specs=[
            pl.BlockSpec((RG, BLK),
                         lambda rg, j: (jnp.minimum(rg, RGN - 1),
                                        jnp.where(rg < RGN, j, NBLK - 1))),
        ],
        out_specs=[
            pl.BlockSpec((RG, BLK),
                         lambda rg, j: (jnp.maximum(rg - 1, 0),
                                        jnp.where(rg > 0, j, 0))),
        ],
        out_shape=[jax.ShapeDtypeStruct((R, C), jnp.float32)],
        scratch_shapes=[
            pltpu.VMEM((RG, 128), jnp.float32),
            pltpu.VMEM((RG, 128), jnp.int32),
            pltpu.VMEM((RG, 128), jnp.int32),
        ],
        compiler_params=pltpu.CompilerParams(
            dimension_semantics=("arbitrary", "arbitrary"),
        ),
    )(logits)[0]


_CHB = 6400
_NFULL = C // _CHB
_SCEND = (C // 128) * 128
_TAILO = _NFULL * _CHB
_TAILW = _SCEND - _TAILO
_TAILV = _TAILW // 16
_UNR = 8

_SC_MESH = dict(core_axis_name="c", subcore_axis_name="s")


def _stripe_bounds(s):
    lo = jnp.where(s < 4, 20 * s, 19 * s + 4)
    hi = jnp.where(s < 3, 20 * (s + 1), 19 * (s + 1) + 4)
    return lo, hi


def _row_exp_sum(buf, r, nvec, acc, unr=_UNR):
    assert nvec % unr == 0
    def step(i, a):
        for u in range(unr):
            a = a + jnp.exp(buf[r, pl.ds(i * (16 * unr) + u * 16, 16)])
        return a
    return lax.fori_loop(0, nvec // unr, step, acc)


def _sc_partial_sums(logits):
    @functools.partial(
        pl.kernel, mesh=plsc.VectorSubcoreMesh(**_SC_MESH),
        out_type=jax.ShapeDtypeStruct((RGN, 8, 8, 16), jnp.float32),
        scratch_types=[
            pltpu.VMEM((RG, _CHB), jnp.float32),
            pltpu.VMEM((RG, 16), jnp.float32),
        ],
    )
    def ka(x_hbm, sums_hbm, buf, accv):
        w = lax.axis_index("s") * 2 + lax.axis_index("c")
        g = w // 8
        s = w % 8
        lo, hi = _stripe_bounds(s)

        def chunk(c, accs):
            pltpu.sync_copy(
                x_hbm.at[pl.ds(g * RG, RG), pl.ds(c * _CHB, _CHB)], buf)
            return tuple(_row_exp_sum(buf, r, _CHB // 16, accs[r])
                         for r in range(RG))

        accs = lax.fori_loop(
            lo, hi, chunk,
            tuple(jnp.zeros((16,), jnp.float32) for _ in range(RG)))

        @pl.when(s == 7)
        def _tail():
            pltpu.sync_copy(
                x_hbm.at[pl.ds(g * RG, RG), pl.ds(_TAILO, _TAILW)],
                buf.at[:, pl.ds(0, _TAILW)])
            for r in range(RG):
                accv[r] = _row_exp_sum(buf, r, _TAILV, accs[r])

        @pl.when(s != 7)
        def _notail():
            for r in range(RG):
                accv[r] = accs[r]

        pltpu.sync_copy(accv, sums_hbm.at[g, s])

    return ka(logits)


def _sc_pi_write(logits, sums):
    @functools.partial(
        pl.kernel, mesh=plsc.VectorSubcoreMesh(**_SC_MESH),
        out_type=jax.ShapeDtypeStruct((R, C), jnp.float32),
        scratch_types=[
            pltpu.VMEM((RG, _CHB), jnp.float32),
            pltpu.VMEM((RG, _CHB), jnp.float32),
            pltpu.VMEM((8, RG, 16), jnp.float32),
        ],
    )
    def kb(x_hbm, sums_hbm, pi_hbm, buf, obuf, sumv):
        w = lax.axis_index("s") * 2 + lax.axis_index("c")
        g = w // 8
        s = w % 8
        pltpu.sync_copy(sums_hbm.at[g], sumv)
        lane16 = lax.iota(jnp.int32, 16)
        recs = []
        for r in range(RG):
            t = sumv[0, r]
            for q in range(1, 8):
                t = t + sumv[q, r]
            dnums = lax.GatherDimensionNumbers(
                offset_dims=(), collapsed_slice_dims=(0,),
                start_index_map=(0,))
            for d in (8, 4, 2, 1):
                t = t + lax.gather(
                    t, (lane16 ^ d)[:, None], dimension_numbers=dnums,
                    slice_sizes=(1,),
                    mode=lax.GatherScatterMode.PROMISE_IN_BOUNDS)
            recs.append(jnp.ones((16,), jnp.float32) / t)

        def norm_rows(nvec, unr=_UNR):
            assert nvec % unr == 0
            def step(i, carry):
                for u in range(unr):
                    off = i * (16 * unr) + u * 16
                    for r in range(RG):
                        obuf[r, pl.ds(off, 16)] = (
                            jnp.exp(buf[r, pl.ds(off, 16)]) * recs[r])
                return carry
            lax.fori_loop(0, nvec // unr, step, 0)

        lo, hi = _stripe_bounds(s)

        def chunk(c, carry):
            src = x_hbm.at[pl.ds(g * RG, RG), pl.ds(c * _CHB, _CHB)]
            pltpu.sync_copy(src, buf)
            norm_rows(_CHB // 16)
            pltpu.sync_copy(
                obuf, pi_hbm.at[pl.ds(g * RG, RG), pl.ds(c * _CHB, _CHB)])
            return carry

        lax.fori_loop(lo, hi, chunk, 0)

        @pl.when(s == 7)
        def _tail():
            pltpu.sync_copy(
                x_hbm.at[pl.ds(g * RG, RG), pl.ds(_TAILO, _TAILW)],
                buf.at[:, pl.ds(0, _TAILW)])
            norm_rows(_TAILV)
            pltpu.sync_copy(
                obuf.at[:, pl.ds(0, _TAILW)],
                pi_hbm.at[pl.ds(g * RG, RG), pl.ds(_TAILO, _TAILW)])

    return kb(logits, sums)


def _tc_tail_body(x_ref, sums_ref, pia_ref, pi_ref):
    del pia_ref
    lane = jax.lax.broadcasted_iota(jnp.int32, (R, 128), 1)
    xm = jnp.where(lane < C - _SCEND, x_ref[...], -jnp.inf)
    e = jnp.exp(xm)
    t64 = jnp.sum(e, axis=1, keepdims=True)
    parts = []
    for g in range(RGN):
        tg = sums_ref[g, 0]
        for s in range(1, 8):
            tg = tg + sums_ref[g, s]
        parts.append(jnp.sum(tg, axis=1, keepdims=True))
    tot = jnp.concatenate(parts, axis=0) + t64
    pi_ref[...] = e * (jnp.float32(1.0) / tot)


def _tc_tail_fix(logits, sums, pi_bulk):
    blk = _SCEND // 128
    return pl.pallas_call(
        _tc_tail_body,
        grid=(1,),
        in_specs=[
            pl.BlockSpec((R, 128), lambda i: (0, blk)),
            pl.BlockSpec((RGN, 8, 8, 16), lambda i: (0, 0, 0, 0)),
            pl.BlockSpec((R, 128), lambda i: (0, blk)),
        ],
        out_specs=[pl.BlockSpec((R, 128), lambda i: (0, blk))],
        out_shape=[jax.ShapeDtypeStruct((R, C), jnp.float32)],
        input_output_aliases={2: 0},
    )(logits, sums, pi_bulk)[0]


def _sc_pi_kernel(logits):
    sums = _sc_partial_sums(logits)
    pi_bulk = _sc_pi_write(logits, sums)
    return _tc_tail_fix(logits, sums, pi_bulk)


def kernel(logits):
    one_hot = _tc_one_hot(logits)
    pi = _sc_pi_kernel(logits)
    return (one_hot, pi)


# speedup vs baseline: 1.2392x; 1.2358x over previous
import functools
import numpy as np
import jax
import jax.numpy as jnp
from jax import lax
from jax.experimental import pallas as pl
from jax.experimental.pallas import tpu as pltpu
from jax.experimental.pallas import tpu_sc as plsc

R, C = 32, 1000000
RG = 8
RGN = R // RG
BLK = 50176
NBLK = (C + BLK - 1) // BLK
TW = 512
NT = BLK // TW

_K0 = np.uint32(928981903)
_K1 = np.uint32(3453687069)
_K2 = np.uint32(_K0 ^ _K1 ^ np.uint32(0x1BD11BDA))
_TINY = np.float32(np.finfo(np.float32).tiny)


def _threefry_bits(x1):
    ks = (jnp.uint32(_K0), jnp.uint32(_K1), jnp.uint32(_K2))
    x0 = jnp.full_like(x1, ks[0])
    rots = ((13, 15, 26, 6), (17, 29, 16, 24))
    for i in range(5):
        for d in rots[i % 2]:
            x0 = x0 + x1
            x1 = ((x1 << jnp.uint32(d)) | (x1 >> jnp.uint32(32 - d))) ^ x0
        x0 = x0 + ks[(i + 1) % 3]
        x1 = x1 + ks[(i + 2) % 3] + jnp.uint32(i + 1)
    return x0 ^ x1


def _tc_body(xc_ref, oh_ref, m_ref, idx_ref, idxp_ref):
    rg = pl.program_id(0)
    j = pl.program_id(1)

    @pl.when(j == 0)
    def _roll():
        idxp_ref[...] = idx_ref[...]
        m_ref[...] = jnp.full((RG, 128), -jnp.inf, jnp.float32)
        idx_ref[...] = jnp.zeros((RG, 128), jnp.int32)

    @pl.when(rg < RGN)
    def _reduce():
        lane_t = jax.lax.broadcasted_iota(jnp.int32, (RG, TW), 1)
        srow = jax.lax.broadcasted_iota(jnp.int32, (RG, TW), 0)
        row_ck = ((rg * RG + srow) * C).astype(jnp.uint32) + jnp.uint32(_K1)
        g_acc = jnp.full((RG, TW), -jnp.inf, jnp.float32)
        i_acc = jnp.zeros((RG, TW), jnp.int32)
        for t in range(NT):
            colt = j * BLK + t * TW + lane_t
            x = xc_ref[:, t * TW:(t + 1) * TW]
            if (NBLK - 1) * BLK + (t + 1) * TW > C:
                x = jnp.where(colt < C, x, -jnp.inf)
            bits = _threefry_bits(row_ck + colt.astype(jnp.uint32))
            fl = jax.lax.bitcast_convert_type(
                (bits >> jnp.uint32(9)) | jnp.uint32(0x3F800000), jnp.float32)
            u = jnp.maximum(fl - jnp.float32(1.0), _TINY)
            gum = -jnp.log(-jnp.log(u))
            g = x + gum
            upd = g > g_acc
            g_acc = jnp.maximum(g_acc, g)
            i_acc = jnp.where(upd, colt, i_acc)

        bm = jnp.max(g_acc, axis=1, keepdims=True)
        cand = jnp.where(g_acc == bm, i_acc, jnp.int32(2**30))
        bidx = jnp.min(cand, axis=1, keepdims=True)

        bm_f = jnp.broadcast_to(bm, (RG, 128))
        bidx_f = jnp.broadcast_to(bidx, (RG, 128))
        upd = bm_f > m_ref[...]
        m_ref[...] = jnp.where(upd, bm_f, m_ref[...])
        idx_ref[...] = jnp.where(upd, bidx_f, idx_ref[...])

    @pl.when(rg > 0)
    def _write():
        lane = jax.lax.broadcasted_iota(jnp.int32, (RG, BLK), 1)
        oh_ref[...] = jnp.where(lane == idxp_ref[:, 0:1] - j * BLK,
                                jnp.float32(1.0), jnp.float32(0.0))


def _tc_one_hot(logits):
    grid = (RGN + 1, NBLK)
    return pl.pallas_call(
        _tc_body,
        grid=grid,
        in_specs=[
            pl.BlockSpec((RG, BLK),
                         lambda rg, j: (jnp.minimum(rg, RGN - 1),
                                        jnp.where(rg < RGN, j, NBLK - 1))),
        ],
        out_specs=[
            pl.BlockSpec((RG, BLK),
                         lambda rg, j: (jnp.maximum(rg - 1, 0),
                                        jnp.where(rg > 0, j, 0))),
        ],
        out_shape=[jax.ShapeDtypeStruct((R, C), jnp.float32)],
        scratch_shapes=[
            pltpu.VMEM((RG, 128), jnp.float32),
            pltpu.VMEM((RG, 128), jnp.int32),
            pltpu.VMEM((RG, 128), jnp.int32),
        ],
        compiler_params=pltpu.CompilerParams(
            dimension_semantics=("arbitrary", "arbitrary"),
        ),
    )(logits)[0]


_CHB = 6400
_NFULL = C // _CHB
_SCEND = (C // 128) * 128
_TAILO = _NFULL * _CHB
_TAILW = _SCEND - _TAILO
_TAILV = _TAILW // 16
_UNR = 8

_SC_MESH = dict(core_axis_name="c", subcore_axis_name="s")


def _stripe_bounds(s):
    lo = jnp.where(s < 4, 20 * s, 19 * s + 4)
    hi = jnp.where(s < 3, 20 * (s + 1), 19 * (s + 1) + 4)
    return lo, hi


def _row_exp_sum(buf, r, nvec, acc, unr=_UNR):
    assert nvec % unr == 0
    def step(i, a):
        for u in range(unr):
            a = a + jnp.exp(buf[r, pl.ds(i * (16 * unr) + u * 16, 16)])
        return a
    return lax.fori_loop(0, nvec // unr, step, acc)


def _sc_pi_bulk(logits):
    @functools.partial(
        pl.kernel, mesh=plsc.VectorSubcoreMesh(**_SC_MESH),
        out_type=[jax.ShapeDtypeStruct((R, C), jnp.float32),
                  jax.ShapeDtypeStruct((R, 16), jnp.float32),
                  jax.ShapeDtypeStruct((RGN, 8, RG, 16), jnp.float32)],
        scratch_types=[
            pltpu.VMEM((RG, _CHB), jnp.float32),
            pltpu.VMEM((RG, _CHB), jnp.float32),
            pltpu.VMEM((RG, 16), jnp.float32),
            pltpu.VMEM((8, RG, 16), jnp.float32),
        ],
    )
    def k(x_hbm, pi_hbm, tot_hbm, sums_hbm, buf, obuf, accv, sumv):
        s16 = lax.axis_index("s")
        c = lax.axis_index("c")
        g = 2 * c + s16 // 8
        s = s16 % 8
        lo, hi = _stripe_bounds(s)
        rows = pl.ds(g * RG, RG)

        def chunk(cc, accs):
            pltpu.sync_copy(x_hbm.at[rows, pl.ds(cc * _CHB, _CHB)], buf)
            return tuple(_row_exp_sum(buf, r, _CHB // 16, accs[r])
                         for r in range(RG))

        accs = lax.fori_loop(
            lo, hi, chunk,
            tuple(jnp.zeros((16,), jnp.float32) for _ in range(RG)))

        @pl.when(s == 7)
        def _tail_sum():
            pltpu.sync_copy(x_hbm.at[rows, pl.ds(_TAILO, _TAILW)],
                            buf.at[:, pl.ds(0, _TAILW)])
            for r in range(RG):
                accv[r] = _row_exp_sum(buf, r, _TAILV, accs[r])

        @pl.when(s != 7)
        def _no_tail():
            for r in range(RG):
                accv[r] = accs[r]

        pltpu.sync_copy(accv, sums_hbm.at[g, s])
        plsc.subcore_barrier()
        pltpu.sync_copy(sums_hbm.at[g], sumv)

        lane16 = lax.iota(jnp.int32, 16)
        dnums = lax.GatherDimensionNumbers(
            offset_dims=(), collapsed_slice_dims=(0,),
            start_index_map=(0,))
        recs = []
        for r in range(RG):
            t = sumv[0, r]
            for q in range(1, 8):
                t = t + sumv[q, r]
            for d in (8, 4, 2, 1):
                t = t + lax.gather(
                    t, (lane16 ^ d)[:, None], dimension_numbers=dnums,
                    slice_sizes=(1,),
                    mode=lax.GatherScatterMode.PROMISE_IN_BOUNDS)
            recs.append(jnp.ones((16,), jnp.float32) / t)

        @pl.when(s == 0)
        def _write_tots():
            for r in range(RG):
                accv[r] = jnp.ones((16,), jnp.float32) / recs[r]
            pltpu.sync_copy(accv, tot_hbm.at[rows])

        def norm_rows(nvec, unr=_UNR):
            assert nvec % unr == 0
            def step(i, carry):
                for u in range(unr):
                    off = i * (16 * unr) + u * 16
                    for r in range(RG):
                        obuf[r, pl.ds(off, 16)] = (
                            jnp.exp(buf[r, pl.ds(off, 16)]) * recs[r])
                return carry
            lax.fori_loop(0, nvec // unr, step, 0)

        def chunk2(cc, carry):
            pltpu.sync_copy(x_hbm.at[rows, pl.ds(cc * _CHB, _CHB)], buf)
            norm_rows(_CHB // 16)
            pltpu.sync_copy(obuf, pi_hbm.at[rows, pl.ds(cc * _CHB, _CHB)])
            return carry

        lax.fori_loop(lo, hi, chunk2, 0)

        @pl.when(s == 7)
        def _tail_pi():
            pltpu.sync_copy(x_hbm.at[rows, pl.ds(_TAILO, _TAILW)],
                            buf.at[:, pl.ds(0, _TAILW)])
            norm_rows(_TAILV)
            pltpu.sync_copy(obuf.at[:, pl.ds(0, _TAILW)],
                            pi_hbm.at[rows, pl.ds(_TAILO, _TAILW)])

    return k(logits)


def _tc_tail_body(x_ref, tot_ref, oh_ref, pia_ref, pi_ref):
    del oh_ref
    del pia_ref
    lane = jax.lax.broadcasted_iota(jnp.int32, (R, 128), 1)
    xm = jnp.where(lane < C - _SCEND, x_ref[...], -jnp.inf)
    e = jnp.exp(xm)
    t64 = jnp.sum(e, axis=1, keepdims=True)
    tot = tot_ref[:, 0:1] + t64
    pi_ref[...] = e * (jnp.float32(1.0) / tot)


def _tc_tail_fix(logits, tots, one_hot, pi_bulk):
    blk = _SCEND // 128
    return pl.pallas_call(
        _tc_tail_body,
        grid=(1,),
        in_specs=[
            pl.BlockSpec((R, 128), lambda i: (0, blk)),
            pl.BlockSpec((R, 16), lambda i: (0, 0)),
            pl.BlockSpec((R, 128), lambda i: (0, 0)),
            pl.BlockSpec((R, 128), lambda i: (0, blk)),
        ],
        out_specs=[pl.BlockSpec((R, 128), lambda i: (0, blk))],
        out_shape=[jax.ShapeDtypeStruct((R, C), jnp.float32)],
        input_output_aliases={3: 0},
    )(logits, tots, one_hot, pi_bulk)[0]


def kernel(logits):
    pi_bulk, tots, _sums_unused = _sc_pi_bulk(logits)
    one_hot = _tc_one_hot(logits)
    pi = _tc_tail_fix(logits, tots, one_hot, pi_bulk)
    return (one_hot, pi)
